# baseline (speedup 1.0000x reference)
import functools

import jax
import jax.numpy as jnp
from jax import lax
from jax.experimental import pallas as pl
from jax.experimental.pallas import tpu as pltpu
from jax.experimental.pallas import tpu_sc as plsc

K_NN = 16


def _topk_body(nkeys, bk, qn_ref, kn_ref, qsq_ref, ksq_ref,
               dist_ref, nbrs_ref, wrep_ref,
               dref, bestd_ref, besti_ref):
    j = pl.program_id(0)
    nblk = pl.num_programs(0)
    q_rows = qn_ref.shape[0]

    @pl.when(j == 0)
    def _init():
        bestd_ref[...] = jnp.full((q_rows, K_NN), jnp.inf, jnp.float32)
        besti_ref[...] = jnp.zeros((q_rows, K_NN), jnp.int32)

    dot = lax.dot_general(qn_ref[...], kn_ref[...], (((1,), (1,)), ((), ())),
                          preferred_element_type=jnp.float32)
    d2 = qsq_ref[...] + ksq_ref[pl.ds(j, 1), :] - 2.0 * dot
    dref[...] = jnp.maximum(d2, 1e-12)

    def body(_):
        dd = dref[...]
        m = jnp.min(dd, axis=1, keepdims=True)
        alive = m < bestd_ref[:, K_NN - 1:K_NN]
        cols = lax.broadcasted_iota(jnp.int32, (q_rows, bk), 1) + j * bk
        sel = dd == m
        idx = jnp.min(jnp.where(sel, cols, jnp.int32(2 ** 30)),
                      axis=1, keepdims=True)
        dref[...] = jnp.where(sel & (cols == idx) & alive, jnp.inf, dd)

        bd = bestd_ref[...]
        bi = besti_ref[...]
        lt = bd <= m
        sb = jnp.concatenate(
            [jnp.full((q_rows, 1), -jnp.inf, jnp.float32), bd[:, :K_NN - 1]],
            axis=1)
        si = jnp.concatenate(
            [jnp.zeros((q_rows, 1), jnp.int32), bi[:, :K_NN - 1]], axis=1)
        slt = sb <= m
        nbd = jnp.where(lt, bd, jnp.where(slt, m, sb))
        nbi = jnp.where(lt, bi, jnp.where(slt, idx, si))
        bestd_ref[...] = jnp.where(alive, nbd, bd)
        besti_ref[...] = jnp.where(alive, nbi, bi)
        return jnp.any(alive).astype(jnp.int32)

    lax.while_loop(lambda go: go != 0, body, jnp.int32(1))

    @pl.when(j == nblk - 1)
    def _finish():
        d = jnp.sqrt(bestd_ref[...])
        dist_ref[...] = d
        nbrs_ref[...] = besti_ref[...]
        w = 1.0 - d
        wrep_ref[...] = jnp.broadcast_to(w[:, :, None], w.shape + (16,))


def _topk_call(queries, keys, bk=4000, interpret=False):
    q_rows, dim = queries.shape
    nkeys = keys.shape[0]
    nblk = nkeys // bk
    assert nblk * bk == nkeys
    qn = queries / jnp.sum(queries, axis=1, keepdims=True)
    kn = keys / jnp.sum(keys, axis=1, keepdims=True)
    qsq = jnp.sum(qn * qn, axis=1, keepdims=True)
    ksq = jnp.sum(kn * kn, axis=1).reshape(nblk, bk)
    body = functools.partial(_topk_body, nkeys, bk)
    return pl.pallas_call(
        body,
        grid=(nblk,),
        in_specs=[
            pl.BlockSpec((q_rows, dim), lambda j: (0, 0)),
            pl.BlockSpec((bk, dim), lambda j: (j, 0)),
            pl.BlockSpec((q_rows, 1), lambda j: (0, 0)),
            pl.BlockSpec((nblk, bk), lambda j: (0, 0)),
        ],
        out_specs=[
            pl.BlockSpec((q_rows, K_NN), lambda j: (0, 0)),
            pl.BlockSpec((q_rows, K_NN), lambda j: (0, 0)),
            pl.BlockSpec((q_rows, K_NN, 16), lambda j: (0, 0, 0)),
        ],
        out_shape=[
            jax.ShapeDtypeStruct((q_rows, K_NN), jnp.float32),
            jax.ShapeDtypeStruct((q_rows, K_NN), jnp.int32),
            jax.ShapeDtypeStruct((q_rows, K_NN, 16), jnp.float32),
        ],
        scratch_shapes=[
            pltpu.VMEM((q_rows, bk), jnp.float32),
            pltpu.VMEM((q_rows, K_NN), jnp.float32),
            pltpu.VMEM((q_rows, K_NN), jnp.int32),
        ],
        interpret=interpret,
    )(qn, kn, qsq, ksq)


def _sc_score_body(q_per_w, dim, nbrs_hbm, wrep_hbm, ctr_hbm, out_hbm,
                   idx_v, w_v, rows_v, out_v, sem):
    nc = 2
    wid = lax.axis_index("s") * nc + lax.axis_index("c")
    p_per_w = q_per_w * K_NN
    base = wid * p_per_w
    pltpu.sync_copy(nbrs_hbm.at[pl.ds(base, p_per_w)], idx_v)
    pltpu.sync_copy(wrep_hbm.at[pl.ds(base, p_per_w)], w_v)
    pltpu.async_copy(ctr_hbm.at[idx_v], rows_v, sem).wait()

    ncb = dim // 16

    def qbody(qi, carry):
        accs = [jnp.zeros((16,), jnp.float32) for _ in range(ncb)]
        for t in range(K_NN):
            w = w_v[qi * K_NN + t, pl.ds(0, 16)]
            for cb in range(ncb):
                r = rows_v[qi * K_NN + t, pl.ds(cb * 16, 16)]
                accs[cb] = accs[cb] + w * r
        for cb in range(ncb):
            out_v[qi, pl.ds(cb * 16, 16)] = accs[cb]
        return carry

    lax.fori_loop(0, q_per_w, qbody, jnp.int32(0))
    pltpu.sync_copy(out_v, out_hbm.at[pl.ds(wid * q_per_w, q_per_w)])


def _sc_score_call(nbrs_flat, wrep_flat, ctr):
    nkeys, dim = ctr.shape
    npairs = nbrs_flat.shape[0]
    nq = npairs // K_NN
    nworkers = 32
    q_per_w = nq // nworkers
    p_per_w = q_per_w * K_NN
    mesh = plsc.VectorSubcoreMesh(core_axis_name="c", subcore_axis_name="s")
    body = functools.partial(_sc_score_body, q_per_w, dim)
    fn = pl.kernel(
        body,
        mesh=mesh,
        compiler_params=pltpu.CompilerParams(use_tc_tiling_on_sc=False),
        out_type=jax.ShapeDtypeStruct((nq, dim), jnp.float32),
        scratch_types=[
            pltpu.VMEM((p_per_w,), jnp.int32),
            pltpu.VMEM((p_per_w, 16), jnp.float32),
            pltpu.VMEM((p_per_w, dim), jnp.float32),
            pltpu.VMEM((q_per_w, dim), jnp.float32),
            pltpu.SemaphoreType.DMA,
        ],
    )
    return fn(nbrs_flat, wrep_flat, ctr)


def _norm_body(s_ref, out_ref):
    s = s_ref[...]
    out_ref[...] = s / jnp.sum(s, axis=1, keepdims=True)


def _norm_call(raw):
    return pl.pallas_call(
        _norm_body,
        out_shape=jax.ShapeDtypeStruct(raw.shape, jnp.float32),
    )(raw)


def kernel(queries, keys, ctr):
    distances, nbrs, wrep = _topk_call(queries, keys)
    raw = _sc_score_call(nbrs.reshape(-1), wrep.reshape(-1, 16), ctr)
    score = _norm_call(raw)
    return score, distances, nbrs


# speedup vs baseline: 1.5682x; 1.5682x over previous
import functools

import jax
import jax.numpy as jnp
from jax import lax
from jax.experimental import pallas as pl
from jax.experimental.pallas import tpu as pltpu
from jax.experimental.pallas import tpu_sc as plsc

K_NN = 16


def _topk_body(nkeys, bk, qn_ref, kn_ref, qsq_ref, ksq_ref,
               dist_ref, nbrs_ref, wrep_ref,
               dref, bestd_ref, besti_ref):
    j = pl.program_id(0)
    nblk = pl.num_programs(0)
    q_rows = qn_ref.shape[0]

    @pl.when(j == 0)
    def _init():
        bestd_ref[...] = jnp.full((q_rows, K_NN), jnp.inf, jnp.float32)
        besti_ref[...] = jnp.zeros((q_rows, K_NN), jnp.int32)

    dot = lax.dot_general(qn_ref[...], kn_ref[...], (((1,), (1,)), ((), ())),
                          preferred_element_type=jnp.float32)
    d2 = qsq_ref[...] + ksq_ref[pl.ds(j, 1), :] - 2.0 * dot
    dref[...] = jnp.maximum(d2, 1e-12)

    def body(_):
        dd = dref[...]
        m = jnp.min(dd, axis=1, keepdims=True)
        alive = m < bestd_ref[:, K_NN - 1:K_NN]
        cols = lax.broadcasted_iota(jnp.int32, (q_rows, bk), 1) + j * bk
        sel = dd == m
        idx = jnp.min(jnp.where(sel, cols, jnp.int32(2 ** 30)),
                      axis=1, keepdims=True)
        dref[...] = jnp.where(sel & (cols == idx) & alive, jnp.inf, dd)

        bd = bestd_ref[...]
        bi = besti_ref[...]
        lt = bd <= m
        sb = jnp.concatenate(
            [jnp.full((q_rows, 1), -jnp.inf, jnp.float32), bd[:, :K_NN - 1]],
            axis=1)
        si = jnp.concatenate(
            [jnp.zeros((q_rows, 1), jnp.int32), bi[:, :K_NN - 1]], axis=1)
        slt = sb <= m
        nbd = jnp.where(lt, bd, jnp.where(slt, m, sb))
        nbi = jnp.where(lt, bi, jnp.where(slt, idx, si))
        bestd_ref[...] = jnp.where(alive, nbd, bd)
        besti_ref[...] = jnp.where(alive, nbi, bi)
        return jnp.any(alive).astype(jnp.int32)

    lax.while_loop(lambda go: go != 0, body, jnp.int32(1))

    @pl.when(j == nblk - 1)
    def _finish():
        d = jnp.sqrt(bestd_ref[...])
        dist_ref[...] = d
        nbrs_ref[...] = besti_ref[...]
        w = 1.0 - d
        wrep_ref[...] = jnp.broadcast_to(w[:, :, None], w.shape + (16,))


def _topk_call(queries, keys, bk=1000, interpret=False):
    q_rows, dim = queries.shape
    nkeys = keys.shape[0]
    nblk = nkeys // bk
    assert nblk * bk == nkeys
    qn = queries / jnp.sum(queries, axis=1, keepdims=True)
    kn = keys / jnp.sum(keys, axis=1, keepdims=True)
    qsq = jnp.sum(qn * qn, axis=1, keepdims=True)
    ksq = jnp.sum(kn * kn, axis=1).reshape(nblk, bk)
    body = functools.partial(_topk_body, nkeys, bk)
    return pl.pallas_call(
        body,
        grid=(nblk,),
        in_specs=[
            pl.BlockSpec((q_rows, dim), lambda j: (0, 0)),
            pl.BlockSpec((bk, dim), lambda j: (j, 0)),
            pl.BlockSpec((q_rows, 1), lambda j: (0, 0)),
            pl.BlockSpec((nblk, bk), lambda j: (0, 0)),
        ],
        out_specs=[
            pl.BlockSpec((q_rows, K_NN), lambda j: (0, 0)),
            pl.BlockSpec((q_rows, K_NN), lambda j: (0, 0)),
            pl.BlockSpec((q_rows, K_NN, 16), lambda j: (0, 0, 0)),
        ],
        out_shape=[
            jax.ShapeDtypeStruct((q_rows, K_NN), jnp.float32),
            jax.ShapeDtypeStruct((q_rows, K_NN), jnp.int32),
            jax.ShapeDtypeStruct((q_rows, K_NN, 16), jnp.float32),
        ],
        scratch_shapes=[
            pltpu.VMEM((q_rows, bk), jnp.float32),
            pltpu.VMEM((q_rows, K_NN), jnp.float32),
            pltpu.VMEM((q_rows, K_NN), jnp.int32),
        ],
        interpret=interpret,
    )(qn, kn, qsq, ksq)


def _sc_score_body(q_per_w, dim, nbrs_hbm, wrep_hbm, ctr_hbm, out_hbm,
                   idx_v, w_v, rows_v, out_v, sem):
    nc = 2
    wid = lax.axis_index("s") * nc + lax.axis_index("c")
    p_per_w = q_per_w * K_NN
    base = wid * p_per_w
    pltpu.sync_copy(nbrs_hbm.at[pl.ds(base, p_per_w)], idx_v)
    pltpu.sync_copy(wrep_hbm.at[pl.ds(base, p_per_w)], w_v)
    pltpu.async_copy(ctr_hbm.at[idx_v], rows_v, sem).wait()

    ncb = dim // 16

    def qbody(qi, carry):
        accs = [jnp.zeros((16,), jnp.float32) for _ in range(ncb)]
        for t in range(K_NN):
            w = w_v[qi * K_NN + t, pl.ds(0, 16)]
            for cb in range(ncb):
                r = rows_v[qi * K_NN + t, pl.ds(cb * 16, 16)]
                accs[cb] = accs[cb] + w * r
        for cb in range(ncb):
            out_v[qi, pl.ds(cb * 16, 16)] = accs[cb]
        return carry

    lax.fori_loop(0, q_per_w, qbody, jnp.int32(0))
    pltpu.sync_copy(out_v, out_hbm.at[pl.ds(wid * q_per_w, q_per_w)])


def _sc_score_call(nbrs_flat, wrep_flat, ctr):
    nkeys, dim = ctr.shape
    npairs = nbrs_flat.shape[0]
    nq = npairs // K_NN
    nworkers = 32
    q_per_w = nq // nworkers
    p_per_w = q_per_w * K_NN
    mesh = plsc.VectorSubcoreMesh(core_axis_name="c", subcore_axis_name="s")
    body = functools.partial(_sc_score_body, q_per_w, dim)
    fn = pl.kernel(
        body,
        mesh=mesh,
        compiler_params=pltpu.CompilerParams(use_tc_tiling_on_sc=False),
        out_type=jax.ShapeDtypeStruct((nq, dim), jnp.float32),
        scratch_types=[
            pltpu.VMEM((p_per_w,), jnp.int32),
            pltpu.VMEM((p_per_w, 16), jnp.float32),
            pltpu.VMEM((p_per_w, dim), jnp.float32),
            pltpu.VMEM((q_per_w, dim), jnp.float32),
            pltpu.SemaphoreType.DMA,
        ],
    )
    return fn(nbrs_flat, wrep_flat, ctr)


def _norm_body(s_ref, out_ref):
    s = s_ref[...]
    out_ref[...] = s / jnp.sum(s, axis=1, keepdims=True)


def _norm_call(raw):
    return pl.pallas_call(
        _norm_body,
        out_shape=jax.ShapeDtypeStruct(raw.shape, jnp.float32),
    )(raw)


def kernel(queries, keys, ctr):
    distances, nbrs, wrep = _topk_call(queries, keys)
    raw = _sc_score_call(nbrs.reshape(-1), wrep.reshape(-1, 16), ctr)
    score = _norm_call(raw)
    return score, distances, nbrs
